# baseline (speedup 1.0000x reference)
import functools

import jax
import jax.numpy as jnp
from jax import lax
from jax.experimental import pallas as pl
from jax.experimental.pallas import tpu as pltpu
from jax.experimental.pallas import tpu_sc as plsc

N = 10000
D = 256
K = 32
KK = 2 * K
NC = 2
NS = 16
NW = NC * NS
NPAD = 10240
NPW = NPAD // NW
NB = 20
NBLK = NPW // NB
NCHUNK = D // 16


NG = KK // 16
CU = 4


def _sc_products_body(pk_hbm, idx_hbm, out_hbm, emb_blk, idx_blk,
                      rows0, rows1, rows2, rows3, rows4, prod_blk, table_sh,
                      sem0, sem1, sem2, sem3, sem4):
    wid = lax.axis_index("s") * NC + lax.axis_index("c")
    base = wid * NPW
    lane = lax.broadcasted_iota(jnp.int32, (16,), 0)
    UN = NB

    sid = lax.axis_index("s")
    tslice = NPAD // NS
    pltpu.sync_copy(pk_hbm.at[pl.ds(sid * tslice, tslice)],
                    table_sh.at[pl.ds(sid * tslice, tslice)])
    plsc.subcore_barrier()

    def gather(u, buf, sem, src):
        return pltpu.async_copy(
            src.at[idx_blk.at[pl.ds(u * KK, KK)]], buf, sem)

    def compute(j, buf):
        zero = jnp.zeros((16,), jnp.float32)

        def grp_body(g, carry3):
            def sub_body(q, res):
                prs = [g * 16 + q * 8 + tt for tt in range(8)]
                acc_a = [zero] * 8
                acc_b = [zero] * 8
                for c in range(NCHUNK // 2):
                    e_pk = plsc.bitcast(emb_blk[j, pl.ds(c * 16, 16)],
                                        jnp.bfloat16)
                    for tt in range(8):
                        pk = plsc.bitcast(buf[prs[tt], pl.ds(c * 16, 16)],
                                          jnp.bfloat16)
                        av, bv = plsc.unpack(
                            pk * e_pk, format=plsc.PackFormat.INTERLEAVED)
                        acc_a[tt] = acc_a[tt] + av
                        acc_b[tt] = acc_b[tt] + bv
                for tt in range(8):
                    res = jnp.where(lane == q * 8 + tt,
                                    jnp.sum(acc_a[tt] + acc_b[tt]), res)
                return res

            res = lax.fori_loop(0, 2, sub_body, zero)
            prod_blk[j, pl.ds(g * 16, 16)] = res
            return carry3

        lax.fori_loop(0, NG, grp_body, 0)

    bufs = None

    def blk_body(blk, carry):
        n0 = base + blk * NB
        pltpu.sync_copy(pk_hbm.at[pl.ds(n0, NB)], emb_blk)
        pltpu.sync_copy(idx_hbm.at[pl.ds(n0 * KK, NB * KK)], idx_blk)
        for b, (buf, sem, src) in enumerate(bufs):
            gather(b, buf, sem, src)

        def unit_body(uu, carry2):
            for b, (buf, sem, src) in enumerate(bufs):
                u = uu * len(bufs) + b
                pltpu.make_async_copy(
                    src.at[idx_blk.at[pl.ds(u * KK, KK)]], buf, sem).wait()
                compute(u, buf)

                @pl.when(u + len(bufs) < UN)
                def _():
                    gather(u + len(bufs), buf, sem, src)
            return carry2

        lax.fori_loop(0, UN // len(bufs), unit_body, 0)
        pltpu.sync_copy(prod_blk, out_hbm.at[pl.ds(n0, NB)])
        return carry

    bufs = [(rows0, sem0, pk_hbm), (rows1, sem1, table_sh),
            (rows2, sem2, table_sh), (rows3, sem3, table_sh),
            (rows4, sem4, table_sh)]
    lax.fori_loop(0, NBLK, blk_body, 0)


_sc_products = functools.partial(
    pl.kernel,
    out_type=jax.ShapeDtypeStruct((NPAD, KK), jnp.float32),
    mesh=plsc.VectorSubcoreMesh(core_axis_name="c", subcore_axis_name="s"),
    compiler_params=pltpu.CompilerParams(
        use_tc_tiling_on_sc=False, needs_layout_passes=False),
    scratch_types=[
        pltpu.VMEM((NB, D // 2), jnp.int32),
        pltpu.VMEM((NB * KK,), jnp.int32),
        pltpu.VMEM((KK, D // 2), jnp.int32),
        pltpu.VMEM((KK, D // 2), jnp.int32),
        pltpu.VMEM((KK, D // 2), jnp.int32),
        pltpu.VMEM((KK, D // 2), jnp.int32),
        pltpu.VMEM((KK, D // 2), jnp.int32),
        pltpu.VMEM((NB, KK), jnp.float32),
        pltpu.VMEM_SHARED((NPAD, D // 2), jnp.int32),
        pltpu.SemaphoreType.DMA,
        pltpu.SemaphoreType.DMA,
        pltpu.SemaphoreType.DMA,
        pltpu.SemaphoreType.DMA,
        pltpu.SemaphoreType.DMA,
    ],
)(_sc_products_body)


def _tc_loss_body(prod_ref, mask_ref, out_ref, acc_ref):
    i = pl.program_id(0)

    @pl.when(i == 0)
    def _():
        acc_ref[0] = 0.0
        acc_ref[1] = 0.0

    x = prod_ref[...]
    m = mask_ref[...]
    sig = jax.nn.sigmoid(x)
    col = lax.broadcasted_iota(jnp.int32, x.shape, 1)
    v = jnp.where(col < K, sig, 1.0 - sig)
    loss_elem = -jnp.log(v + 1e-15)
    acc_ref[0] += jnp.sum(loss_elem * m) / K
    acc_ref[1] += jnp.sum(m)

    @pl.when(i == pl.num_programs(0) - 1)
    def _():
        out_ref[0, 0] = acc_ref[0] / acc_ref[1]


def _tc_loss(products, mask2d):
    rows_per_step = 1024
    grid = (NPAD // rows_per_step,)
    return pl.pallas_call(
        _tc_loss_body,
        grid=grid,
        in_specs=[
            pl.BlockSpec((rows_per_step, KK), lambda i: (i, 0)),
            pl.BlockSpec((rows_per_step, 1), lambda i: (i, 0)),
        ],
        out_specs=pl.BlockSpec((1, 1), lambda i: (0, 0),
                               memory_space=pltpu.SMEM),
        out_shape=jax.ShapeDtypeStruct((1, 1), jnp.float32),
        scratch_shapes=[pltpu.SMEM((2,), jnp.float32)],
    )(products, mask2d)


def kernel(embeddings, neighbors_array, negative_array, mask_array):
    emb_p = jnp.pad(embeddings, ((0, NPAD - N), (0, 0)))
    emb_pk = lax.bitcast_convert_type(
        emb_p.astype(jnp.bfloat16).reshape(NPAD, D // 2, 2), jnp.int32)
    idx_all = jnp.concatenate([neighbors_array, negative_array], axis=1)
    idx_p = jnp.pad(idx_all, ((0, NPAD - N), (0, 0))).reshape(-1)
    mask2d = jnp.pad(mask_array, (0, NPAD - N)).reshape(NPAD, 1)
    products = _sc_products(emb_pk, idx_p)
    loss = _tc_loss(products, mask2d)
    return loss[0, 0]


# speedup vs baseline: 1.1501x; 1.1501x over previous
import functools

import jax
import jax.numpy as jnp
from jax import lax
from jax.experimental import pallas as pl
from jax.experimental.pallas import tpu as pltpu
from jax.experimental.pallas import tpu_sc as plsc

N = 10000
D = 256
K = 32
KK = 2 * K
NC = 2
NS = 16
NW = NC * NS
NPAD = 10240
NPW = NPAD // NW
NB = 32
NBLK = NPW // NB
NCHUNK = D // 16


NG = KK // 16
CU = 4


def _sc_products_body(pk_hbm, idx_hbm, out_hbm, emb_blk, idx_blk,
                      rows0, rows1, rows2, rows3, prod_blk, table_sh,
                      sem0, sem1, sem2, sem3):
    wid = lax.axis_index("s") * NC + lax.axis_index("c")
    base = wid * NPW
    lane = lax.broadcasted_iota(jnp.int32, (16,), 0)
    UN = NB

    sid = lax.axis_index("s")
    tslice = NPAD // NS
    pltpu.sync_copy(pk_hbm.at[pl.ds(sid * tslice, tslice)],
                    table_sh.at[pl.ds(sid * tslice, tslice)])
    plsc.subcore_barrier()

    def gather(u, buf, sem):
        return pltpu.async_copy(
            table_sh.at[idx_blk.at[pl.ds(u * KK, KK)]], buf, sem)

    def compute(j, buf):
        zero = jnp.zeros((16,), jnp.float32)

        def grp_body(g, carry3):
            def sub_body(q, res):
                prs = [g * 16 + q * 8 + tt for tt in range(8)]
                acc_a = [zero] * 8
                acc_b = [zero] * 8
                for c in range(NCHUNK // 2):
                    e_pk = plsc.bitcast(emb_blk[j, pl.ds(c * 16, 16)],
                                        jnp.bfloat16)
                    for tt in range(8):
                        pk = plsc.bitcast(buf[prs[tt], pl.ds(c * 16, 16)],
                                          jnp.bfloat16)
                        av, bv = plsc.unpack(
                            pk * e_pk, format=plsc.PackFormat.INTERLEAVED)
                        acc_a[tt] = acc_a[tt] + av
                        acc_b[tt] = acc_b[tt] + bv
                for tt in range(8):
                    res = jnp.where(lane == q * 8 + tt,
                                    jnp.sum(acc_a[tt] + acc_b[tt]), res)
                return res

            res = lax.fori_loop(0, 2, sub_body, zero)
            prod_blk[j, pl.ds(g * 16, 16)] = res
            return carry3

        lax.fori_loop(0, NG, grp_body, 0)

    bufs = None

    def blk_body(blk, carry):
        n0 = base + blk * NB
        pltpu.sync_copy(pk_hbm.at[pl.ds(n0, NB)], emb_blk)
        pltpu.sync_copy(idx_hbm.at[pl.ds(n0 * KK, NB * KK)], idx_blk)
        for b, (buf, sem) in enumerate(bufs):
            gather(b, buf, sem)

        def unit_body(uu, carry2):
            for b, (buf, sem) in enumerate(bufs):
                u = uu * len(bufs) + b
                pltpu.make_async_copy(
                    table_sh.at[idx_blk.at[pl.ds(u * KK, KK)]],
                    buf, sem).wait()
                compute(u, buf)

                @pl.when(u + len(bufs) < UN)
                def _():
                    gather(u + len(bufs), buf, sem)
            return carry2

        lax.fori_loop(0, UN // len(bufs), unit_body, 0)
        pltpu.sync_copy(prod_blk, out_hbm.at[pl.ds(n0, NB)])
        return carry

    bufs = [(rows0, sem0), (rows1, sem1), (rows2, sem2), (rows3, sem3)]
    lax.fori_loop(0, NBLK, blk_body, 0)


_sc_products = functools.partial(
    pl.kernel,
    out_type=jax.ShapeDtypeStruct((NPAD, KK), jnp.float32),
    mesh=plsc.VectorSubcoreMesh(core_axis_name="c", subcore_axis_name="s"),
    compiler_params=pltpu.CompilerParams(
        use_tc_tiling_on_sc=False, needs_layout_passes=False),
    scratch_types=[
        pltpu.VMEM((NB, D // 2), jnp.int32),
        pltpu.VMEM((NB * KK,), jnp.int32),
        pltpu.VMEM((KK, D // 2), jnp.int32),
        pltpu.VMEM((KK, D // 2), jnp.int32),
        pltpu.VMEM((KK, D // 2), jnp.int32),
        pltpu.VMEM((KK, D // 2), jnp.int32),
        pltpu.VMEM((NB, KK), jnp.float32),
        pltpu.VMEM_SHARED((NPAD, D // 2), jnp.int32),
        pltpu.SemaphoreType.DMA,
        pltpu.SemaphoreType.DMA,
        pltpu.SemaphoreType.DMA,
        pltpu.SemaphoreType.DMA,
    ],
)(_sc_products_body)


def _tc_loss_body(prod_ref, mask_ref, out_ref, acc_ref):
    i = pl.program_id(0)

    @pl.when(i == 0)
    def _():
        acc_ref[0] = 0.0
        acc_ref[1] = 0.0

    x = prod_ref[...]
    m = mask_ref[...]
    sig = jax.nn.sigmoid(x)
    col = lax.broadcasted_iota(jnp.int32, x.shape, 1)
    v = jnp.where(col < K, sig, 1.0 - sig)
    loss_elem = -jnp.log(v + 1e-15)
    acc_ref[0] += jnp.sum(loss_elem * m) / K
    acc_ref[1] += jnp.sum(m)

    @pl.when(i == pl.num_programs(0) - 1)
    def _():
        out_ref[0, 0] = acc_ref[0] / acc_ref[1]


def _tc_loss(products, mask2d):
    rows_per_step = 1024
    grid = (NPAD // rows_per_step,)
    return pl.pallas_call(
        _tc_loss_body,
        grid=grid,
        in_specs=[
            pl.BlockSpec((rows_per_step, KK), lambda i: (i, 0)),
            pl.BlockSpec((rows_per_step, 1), lambda i: (i, 0)),
        ],
        out_specs=pl.BlockSpec((1, 1), lambda i: (0, 0),
                               memory_space=pltpu.SMEM),
        out_shape=jax.ShapeDtypeStruct((1, 1), jnp.float32),
        scratch_shapes=[pltpu.SMEM((2,), jnp.float32)],
    )(products, mask2d)


def kernel(embeddings, neighbors_array, negative_array, mask_array):
    emb_p = jnp.pad(embeddings, ((0, NPAD - N), (0, 0)))
    emb_pk = lax.bitcast_convert_type(
        emb_p.astype(jnp.bfloat16).reshape(NPAD, D // 2, 2), jnp.int32)
    idx_all = jnp.concatenate([neighbors_array, negative_array], axis=1)
    idx_p = jnp.pad(idx_all, ((0, NPAD - N), (0, 0))).reshape(-1)
    mask2d = jnp.pad(mask_array, (0, NPAD - N)).reshape(NPAD, 1)
    products = _sc_products(emb_pk, idx_p)
    loss = _tc_loss(products, mask2d)
    return loss[0, 0]
